# baseline (speedup 1.0000x reference)
import functools

import jax
import jax.numpy as jnp
from jax import lax
from jax.experimental import pallas as pl
from jax.experimental.pallas import tpu as pltpu
from jax.experimental.pallas import tpu_sc as plsc

N_BINS = 128
L = 16
N_CHUNKS = N_BINS // L


def _loss_cdf_body(t_hbm, lt_hbm, lu_hbm, out_hbm,
                   lt_v, lu_v, et_v, a_v, b_v, t_v, u_v, sem):
    nc = 2
    wid = lax.axis_index("s") * nc + lax.axis_index("c")
    r0 = 8 * (wid // 16)
    c0 = 256 * (wid % 16)

    t_cp = pltpu.async_copy(
        t_hbm.at[pl.ds(r0, 8), pl.ds(c0, 256)], t_v, sem)
    pltpu.sync_copy(lt_hbm, lt_v)
    pltpu.sync_copy(lu_hbm, lu_v)

    ex_t = [jnp.exp(lt_v[pl.ds(c * L, L)]) for c in range(N_CHUNKS)]
    ex_u = [jnp.exp(lu_v[pl.ds(c * L, L)]) for c in range(N_CHUNKS)]
    s_t = functools.reduce(lax.add, [jnp.sum(e) for e in ex_t])
    q_t = [e / s_t + 0.001 for e in ex_t]
    z_t = functools.reduce(lax.add, [jnp.sum(q) for q in q_t])
    q_u = [e + 0.001 for e in ex_u]
    z_u = functools.reduce(lax.add, [jnp.sum(q) for q in q_u])

    carry_t = jnp.float32(0.0)
    carry_u = jnp.float32(0.0)
    for c in range(N_CHUNKS):
        w_t = q_t[c] / z_t
        w_u = q_u[c] / z_u
        b = w_u / w_t
        incl_t = plsc.cumsum(w_t)
        incl_u = plsc.cumsum(w_u)
        e_t = incl_t - w_t + carry_t
        e_u = incl_u - w_u + carry_u
        b_v[pl.ds(c * L, L)] = b
        et_v[pl.ds(c * L, L)] = e_t
        a_v[pl.ds(c * L, L)] = e_u - b * e_t
        carry_t = carry_t + jnp.squeeze(lax.slice(incl_t, (L - 1,), (L,)))
        carry_u = carry_u + jnp.squeeze(lax.slice(incl_u, (L - 1,), (L,)))

    t_cp.wait()

    def _process(off):
        r = jax.lax.shift_right_logical(off, 8)
        col = jax.lax.bitwise_and(off, 255)
        tv = t_v[r, pl.ds(col, L)]
        idx = jnp.zeros((L,), jnp.int32)
        for s in (64, 32, 16, 8, 4, 2, 1):
            cand = idx + s
            ev = plsc.load_gather(et_v, [cand])
            idx = jnp.where(ev <= tv, cand, idx)
        av = plsc.load_gather(a_v, [idx])
        bv = plsc.load_gather(b_v, [idx])
        u_v[r, pl.ds(col, L)] = av + bv * tv

    plsc.parallel_loop(0, 1024, L, unroll=4)(_process)
    u_cp = pltpu.async_copy(
        u_v.at[pl.ds(0, 4)], out_hbm.at[pl.ds(r0, 4), pl.ds(c0, 256)], sem)
    plsc.parallel_loop(1024, 2048, L, unroll=4)(_process)
    u_cp.wait()
    pltpu.sync_copy(
        u_v.at[pl.ds(4, 4)], out_hbm.at[pl.ds(r0 + 4, 4), pl.ds(c0, 256)])


def kernel(t, l_t, l_u):
    b, s = t.shape
    mesh = plsc.VectorSubcoreMesh(core_axis_name="c", subcore_axis_name="s")
    run = pl.kernel(
        _loss_cdf_body,
        mesh=mesh,
        compiler_params=pltpu.CompilerParams(
            needs_layout_passes=False, use_tc_tiling_on_sc=True),
        out_type=jax.ShapeDtypeStruct((b, s), jnp.float32),
        scratch_types=[
            pltpu.VMEM((N_BINS,), jnp.float32),
            pltpu.VMEM((N_BINS,), jnp.float32),
            pltpu.VMEM((N_BINS,), jnp.float32),
            pltpu.VMEM((N_BINS,), jnp.float32),
            pltpu.VMEM((N_BINS,), jnp.float32),
            pltpu.VMEM((8, 256), jnp.float32),
            pltpu.VMEM((8, 256), jnp.float32),
            pltpu.SemaphoreType.DMA,
        ],
    )
    return run(t, l_t, l_u)


# speedup vs baseline: 1.0069x; 1.0069x over previous
import functools

import jax
import jax.numpy as jnp
from jax import lax
from jax.experimental import pallas as pl
from jax.experimental.pallas import tpu as pltpu
from jax.experimental.pallas import tpu_sc as plsc

N_BINS = 128
L = 16
N_CHUNKS = N_BINS // L


def _loss_cdf_body(t_hbm, lt_hbm, lu_hbm, out_hbm,
                   lt_v, lu_v, et_v, a_v, b_v, t_v, u_v, sem):
    nc = 2
    wid = lax.axis_index("s") * nc + lax.axis_index("c")
    r0 = 8 * (wid // 16)
    c0 = 256 * (wid % 16)

    t_cp = pltpu.async_copy(
        t_hbm.at[pl.ds(r0, 8), pl.ds(c0, 256)], t_v, sem)
    pltpu.sync_copy(lt_hbm, lt_v)
    pltpu.sync_copy(lu_hbm, lu_v)

    ex_t = [jnp.exp(lt_v[pl.ds(c * L, L)]) for c in range(N_CHUNKS)]
    ex_u = [jnp.exp(lu_v[pl.ds(c * L, L)]) for c in range(N_CHUNKS)]
    s_t = functools.reduce(lax.add, [jnp.sum(e) for e in ex_t])
    q_t = [e / s_t + 0.001 for e in ex_t]
    z_t = functools.reduce(lax.add, [jnp.sum(q) for q in q_t])
    q_u = [e + 0.001 for e in ex_u]
    z_u = functools.reduce(lax.add, [jnp.sum(q) for q in q_u])

    carry_t = jnp.float32(0.0)
    carry_u = jnp.float32(0.0)
    for c in range(N_CHUNKS):
        w_t = q_t[c] / z_t
        w_u = q_u[c] / z_u
        b = w_u / w_t
        incl_t = plsc.cumsum(w_t)
        incl_u = plsc.cumsum(w_u)
        e_t = incl_t - w_t + carry_t
        e_u = incl_u - w_u + carry_u
        b_v[pl.ds(c * L, L)] = b
        et_v[pl.ds(c * L, L)] = e_t
        a_v[pl.ds(c * L, L)] = e_u - b * e_t
        carry_t = carry_t + jnp.squeeze(lax.slice(incl_t, (L - 1,), (L,)))
        carry_u = carry_u + jnp.squeeze(lax.slice(incl_u, (L - 1,), (L,)))

    t_cp.wait()

    def _process(off):
        r = jax.lax.shift_right_logical(off, 8)
        col = jax.lax.bitwise_and(off, 255)
        tv = t_v[r, pl.ds(col, L)]
        idx = jnp.zeros((L,), jnp.int32)
        for s in (64, 32, 16, 8, 4, 2, 1):
            cand = idx + s
            ev = plsc.load_gather(et_v, [cand])
            idx = jnp.where(ev <= tv, cand, idx)
        av = plsc.load_gather(a_v, [idx])
        bv = plsc.load_gather(b_v, [idx])
        u_v[r, pl.ds(col, L)] = av + bv * tv

    plsc.parallel_loop(0, 2048, L, unroll=8)(_process)
    pltpu.sync_copy(u_v, out_hbm.at[pl.ds(r0, 8), pl.ds(c0, 256)])


def kernel(t, l_t, l_u):
    b, s = t.shape
    mesh = plsc.VectorSubcoreMesh(core_axis_name="c", subcore_axis_name="s")
    run = pl.kernel(
        _loss_cdf_body,
        mesh=mesh,
        compiler_params=pltpu.CompilerParams(
            needs_layout_passes=False, use_tc_tiling_on_sc=True),
        out_type=jax.ShapeDtypeStruct((b, s), jnp.float32),
        scratch_types=[
            pltpu.VMEM((N_BINS,), jnp.float32),
            pltpu.VMEM((N_BINS,), jnp.float32),
            pltpu.VMEM((N_BINS,), jnp.float32),
            pltpu.VMEM((N_BINS,), jnp.float32),
            pltpu.VMEM((N_BINS,), jnp.float32),
            pltpu.VMEM((8, 256), jnp.float32),
            pltpu.VMEM((8, 256), jnp.float32),
            pltpu.SemaphoreType.DMA,
        ],
    )
    return run(t, l_t, l_u)


# speedup vs baseline: 1.0211x; 1.0141x over previous
import functools

import jax
import jax.numpy as jnp
from jax import lax
from jax.experimental import pallas as pl
from jax.experimental.pallas import tpu as pltpu
from jax.experimental.pallas import tpu_sc as plsc

N_BINS = 128
L = 16
N_CHUNKS = N_BINS // L


def _loss_cdf_body(t_hbm, lt_hbm, lu_hbm, out_hbm,
                   lt_v, lu_v, et_v, a_v, b_v, t_v, u_v, sem):
    nc = 2
    wid = lax.axis_index("s") * nc + lax.axis_index("c")
    r0 = 8 * (wid // 16)
    c0 = 256 * (wid % 16)

    t_cp = pltpu.async_copy(
        t_hbm.at[pl.ds(r0, 8), pl.ds(c0, 256)], t_v, sem)
    pltpu.sync_copy(lt_hbm, lt_v)
    pltpu.sync_copy(lu_hbm, lu_v)

    ex_t = [jnp.exp(lt_v[pl.ds(c * L, L)]) for c in range(N_CHUNKS)]
    ex_u = [jnp.exp(lu_v[pl.ds(c * L, L)]) for c in range(N_CHUNKS)]
    s_t = functools.reduce(lax.add, [jnp.sum(e) for e in ex_t])
    q_t = [e / s_t + 0.001 for e in ex_t]
    z_t = functools.reduce(lax.add, [jnp.sum(q) for q in q_t])
    q_u = [e + 0.001 for e in ex_u]
    z_u = functools.reduce(lax.add, [jnp.sum(q) for q in q_u])

    carry_t = jnp.float32(0.0)
    carry_u = jnp.float32(0.0)
    for c in range(N_CHUNKS):
        w_t = q_t[c] / z_t
        w_u = q_u[c] / z_u
        b = w_u / w_t
        incl_t = plsc.cumsum(w_t)
        incl_u = plsc.cumsum(w_u)
        e_t = incl_t - w_t + carry_t
        e_u = incl_u - w_u + carry_u
        b_v[pl.ds(c * L, L)] = b
        et_v[pl.ds(c * L, L)] = e_t
        a_v[pl.ds(c * L, L)] = e_u - b * e_t
        carry_t = carry_t + jnp.squeeze(lax.slice(incl_t, (L - 1,), (L,)))
        carry_u = carry_u + jnp.squeeze(lax.slice(incl_u, (L - 1,), (L,)))

    t_cp.wait()

    def _process(off):
        r = jax.lax.shift_right_logical(off, 8)
        col = jax.lax.bitwise_and(off, 255)
        tv = t_v[r, pl.ds(col, L)]
        idx = jnp.zeros((L,), jnp.int32)
        for s in (64, 32, 16, 8, 4, 2, 1):
            cand = idx + s
            ev = plsc.load_gather(et_v, [cand])
            idx = jnp.where(ev <= tv, cand, idx)
        av = plsc.load_gather(a_v, [idx])
        bv = plsc.load_gather(b_v, [idx])
        u_v[r, pl.ds(col, L)] = av + bv * tv

    plsc.parallel_loop(0, 2048, L, unroll=4)(_process)
    pltpu.sync_copy(u_v, out_hbm.at[pl.ds(r0, 8), pl.ds(c0, 256)])


def kernel(t, l_t, l_u):
    b, s = t.shape
    mesh = plsc.VectorSubcoreMesh(core_axis_name="c", subcore_axis_name="s")
    run = pl.kernel(
        _loss_cdf_body,
        mesh=mesh,
        compiler_params=pltpu.CompilerParams(
            needs_layout_passes=False, use_tc_tiling_on_sc=True),
        out_type=jax.ShapeDtypeStruct((b, s), jnp.float32),
        scratch_types=[
            pltpu.VMEM((N_BINS,), jnp.float32),
            pltpu.VMEM((N_BINS,), jnp.float32),
            pltpu.VMEM((N_BINS,), jnp.float32),
            pltpu.VMEM((N_BINS,), jnp.float32),
            pltpu.VMEM((N_BINS,), jnp.float32),
            pltpu.VMEM((8, 256), jnp.float32),
            pltpu.VMEM((8, 256), jnp.float32),
            pltpu.SemaphoreType.DMA,
        ],
    )
    return run(t, l_t, l_u)


# speedup vs baseline: 1.0391x; 1.0176x over previous
import functools

import jax
import jax.numpy as jnp
from jax import lax
from jax.experimental import pallas as pl
from jax.experimental.pallas import tpu as pltpu
from jax.experimental.pallas import tpu_sc as plsc

N_BINS = 128
L = 16
N_CHUNKS = N_BINS // L


def _loss_cdf_body(t_hbm, lt_hbm, lu_hbm, out_hbm,
                   lt_v, lu_v, et_v, a_v, b_v, t_v, u_v, sem, lsem):
    nc = 2
    wid = lax.axis_index("s") * nc + lax.axis_index("c")
    r0 = 8 * (wid // 16)
    c0 = 256 * (wid % 16)

    lt_cp = pltpu.async_copy(lt_hbm, lt_v, lsem)
    lu_cp = pltpu.async_copy(lu_hbm, lu_v, lsem)
    t_cp = pltpu.async_copy(
        t_hbm.at[pl.ds(r0, 8), pl.ds(c0, 256)], t_v, sem)
    lt_cp.wait()
    lu_cp.wait()

    ex_t = [jnp.exp(lt_v[pl.ds(c * L, L)]) for c in range(N_CHUNKS)]
    ex_u = [jnp.exp(lu_v[pl.ds(c * L, L)]) for c in range(N_CHUNKS)]
    s_t = functools.reduce(lax.add, [jnp.sum(e) for e in ex_t])
    q_t = [e / s_t + 0.001 for e in ex_t]
    z_t = functools.reduce(lax.add, [jnp.sum(q) for q in q_t])
    q_u = [e + 0.001 for e in ex_u]
    z_u = functools.reduce(lax.add, [jnp.sum(q) for q in q_u])

    carry_t = jnp.float32(0.0)
    carry_u = jnp.float32(0.0)
    for c in range(N_CHUNKS):
        w_t = q_t[c] / z_t
        w_u = q_u[c] / z_u
        b = w_u / w_t
        incl_t = plsc.cumsum(w_t)
        incl_u = plsc.cumsum(w_u)
        e_t = incl_t - w_t + carry_t
        e_u = incl_u - w_u + carry_u
        b_v[pl.ds(c * L, L)] = b
        et_v[pl.ds(c * L, L)] = e_t
        a_v[pl.ds(c * L, L)] = e_u - b * e_t
        carry_t = carry_t + jnp.squeeze(lax.slice(incl_t, (L - 1,), (L,)))
        carry_u = carry_u + jnp.squeeze(lax.slice(incl_u, (L - 1,), (L,)))

    t_cp.wait()

    def _process(off):
        r = jax.lax.shift_right_logical(off, 8)
        col = jax.lax.bitwise_and(off, 255)
        tv = t_v[r, pl.ds(col, L)]
        idx = jnp.zeros((L,), jnp.int32)
        for s in (64, 32, 16, 8, 4, 2, 1):
            cand = idx + s
            ev = plsc.load_gather(et_v, [cand])
            idx = jnp.where(ev <= tv, cand, idx)
        av = plsc.load_gather(a_v, [idx])
        bv = plsc.load_gather(b_v, [idx])
        u_v[r, pl.ds(col, L)] = av + bv * tv

    plsc.parallel_loop(0, 2048, L, unroll=4)(_process)
    pltpu.sync_copy(u_v, out_hbm.at[pl.ds(r0, 8), pl.ds(c0, 256)])


def kernel(t, l_t, l_u):
    b, s = t.shape
    mesh = plsc.VectorSubcoreMesh(core_axis_name="c", subcore_axis_name="s")
    run = pl.kernel(
        _loss_cdf_body,
        mesh=mesh,
        compiler_params=pltpu.CompilerParams(
            needs_layout_passes=False, use_tc_tiling_on_sc=True),
        out_type=jax.ShapeDtypeStruct((b, s), jnp.float32),
        scratch_types=[
            pltpu.VMEM((N_BINS,), jnp.float32),
            pltpu.VMEM((N_BINS,), jnp.float32),
            pltpu.VMEM((N_BINS,), jnp.float32),
            pltpu.VMEM((N_BINS,), jnp.float32),
            pltpu.VMEM((N_BINS,), jnp.float32),
            pltpu.VMEM((8, 256), jnp.float32),
            pltpu.VMEM((8, 256), jnp.float32),
            pltpu.SemaphoreType.DMA,
            pltpu.SemaphoreType.DMA,
        ],
    )
    return run(t, l_t, l_u)


# speedup vs baseline: 1.0428x; 1.0035x over previous
import functools

import jax
import jax.numpy as jnp
from jax import lax
from jax.experimental import pallas as pl
from jax.experimental.pallas import tpu as pltpu
from jax.experimental.pallas import tpu_sc as plsc

N_BINS = 128
L = 16
N_CHUNKS = N_BINS // L


def _loss_cdf_body(t_hbm, lt_hbm, lu_hbm, out_hbm,
                   lt_v, lu_v, et_v, a_v, b_v, t_v, u_v, sem, lsem):
    nc = 2
    wid = lax.axis_index("s") * nc + lax.axis_index("c")
    r0 = 8 * (wid // 16)
    c0 = 256 * (wid % 16)

    lt_cp = pltpu.async_copy(lt_hbm, lt_v, lsem)
    lu_cp = pltpu.async_copy(lu_hbm, lu_v, lsem)
    t_cp = pltpu.async_copy(
        t_hbm.at[pl.ds(r0, 8), pl.ds(c0, 256)], t_v, sem)
    lt_cp.wait()
    lu_cp.wait()

    scope_tables = jax.named_scope("sc_tables")
    scope_tables.__enter__()
    ex_t = [jnp.exp(lt_v[pl.ds(c * L, L)]) for c in range(N_CHUNKS)]
    ex_u = [jnp.exp(lu_v[pl.ds(c * L, L)]) for c in range(N_CHUNKS)]
    s_t = functools.reduce(lax.add, [jnp.sum(e) for e in ex_t])
    q_t = [e / s_t + 0.001 for e in ex_t]
    z_t = functools.reduce(lax.add, [jnp.sum(q) for q in q_t])
    q_u = [e + 0.001 for e in ex_u]
    z_u = functools.reduce(lax.add, [jnp.sum(q) for q in q_u])

    carry_t = jnp.float32(0.0)
    carry_u = jnp.float32(0.0)
    for c in range(N_CHUNKS):
        w_t = q_t[c] / z_t
        w_u = q_u[c] / z_u
        b = w_u / w_t
        incl_t = plsc.cumsum(w_t)
        incl_u = plsc.cumsum(w_u)
        e_t = incl_t - w_t + carry_t
        e_u = incl_u - w_u + carry_u
        b_v[pl.ds(c * L, L)] = b
        et_v[pl.ds(c * L, L)] = e_t
        a_v[pl.ds(c * L, L)] = e_u - b * e_t
        carry_t = carry_t + jnp.squeeze(lax.slice(incl_t, (L - 1,), (L,)))
        carry_u = carry_u + jnp.squeeze(lax.slice(incl_u, (L - 1,), (L,)))

    scope_tables.__exit__(None, None, None)

    with jax.named_scope("sc_twait"):
        t_cp.wait()

    def _process(off):
        r = jax.lax.shift_right_logical(off, 8)
        col = jax.lax.bitwise_and(off, 255)
        tv = t_v[r, pl.ds(col, L)]
        idx = jnp.zeros((L,), jnp.int32)
        for s in (64, 32, 16, 8, 4, 2, 1):
            cand = idx + s
            ev = plsc.load_gather(et_v, [cand])
            idx = jnp.where(ev <= tv, cand, idx)
        av = plsc.load_gather(a_v, [idx])
        bv = plsc.load_gather(b_v, [idx])
        u_v[r, pl.ds(col, L)] = av + bv * tv

    with jax.named_scope("sc_loop"):
        plsc.parallel_loop(0, 2048, L, unroll=4)(_process)
    with jax.named_scope("sc_drain"):
        pltpu.sync_copy(u_v, out_hbm.at[pl.ds(r0, 8), pl.ds(c0, 256)])


def kernel(t, l_t, l_u):
    b, s = t.shape
    mesh = plsc.VectorSubcoreMesh(core_axis_name="c", subcore_axis_name="s")
    run = pl.kernel(
        _loss_cdf_body,
        mesh=mesh,
        compiler_params=pltpu.CompilerParams(
            needs_layout_passes=False, use_tc_tiling_on_sc=True),
        out_type=jax.ShapeDtypeStruct((b, s), jnp.float32),
        scratch_types=[
            pltpu.VMEM((N_BINS,), jnp.float32),
            pltpu.VMEM((N_BINS,), jnp.float32),
            pltpu.VMEM((N_BINS,), jnp.float32),
            pltpu.VMEM((N_BINS,), jnp.float32),
            pltpu.VMEM((N_BINS,), jnp.float32),
            pltpu.VMEM((8, 256), jnp.float32),
            pltpu.VMEM((8, 256), jnp.float32),
            pltpu.SemaphoreType.DMA,
            pltpu.SemaphoreType.DMA,
        ],
    )
    return run(t, l_t, l_u)


# speedup vs baseline: 1.1537x; 1.1064x over previous
import functools

import jax
import jax.numpy as jnp
from jax import lax
from jax.experimental import pallas as pl
from jax.experimental.pallas import tpu as pltpu
from jax.experimental.pallas import tpu_sc as plsc

N_BINS = 128
L = 16
N_CHUNKS = N_BINS // L


def _loss_cdf_body(t_hbm, lt_hbm, lu_hbm, out_hbm,
                   lt_v, lu_v, et_v, a_v, b_v, t_v, u_v, sem, lsem):
    nc = 2
    wid = lax.axis_index("s") * nc + lax.axis_index("c")
    r0 = 8 * (wid // 16)
    c0 = 256 * (wid % 16)

    lt_cp = pltpu.async_copy(lt_hbm, lt_v, lsem)
    lu_cp = pltpu.async_copy(lu_hbm, lu_v, lsem)
    t_cp = pltpu.async_copy(
        t_hbm.at[pl.ds(r0, 8), pl.ds(c0, 256)], t_v, sem)
    lt_cp.wait()
    lu_cp.wait()

    ex_t = [jnp.exp(lt_v[pl.ds(c * L, L)]) for c in range(N_CHUNKS)]
    ex_u = [jnp.exp(lu_v[pl.ds(c * L, L)]) for c in range(N_CHUNKS)]
    s_t = functools.reduce(lax.add, [jnp.sum(e) for e in ex_t])
    q_t = [e / s_t + 0.001 for e in ex_t]
    z_t = functools.reduce(lax.add, [jnp.sum(q) for q in q_t])
    q_u = [e + 0.001 for e in ex_u]
    z_u = functools.reduce(lax.add, [jnp.sum(q) for q in q_u])

    carry_t = jnp.float32(0.0)
    carry_u = jnp.float32(0.0)
    carries_t = []
    for c in range(N_CHUNKS):
        w_t = q_t[c] / z_t
        w_u = q_u[c] / z_u
        b = w_u / w_t
        incl_t = plsc.cumsum(w_t)
        incl_u = plsc.cumsum(w_u)
        e_t = incl_t - w_t + carry_t
        e_u = incl_u - w_u + carry_u
        b_v[pl.ds(17 * c, L)] = b
        et_v[pl.ds(17 * c, L)] = e_t
        a_v[pl.ds(17 * c, L)] = e_u - b * e_t
        carry_t = carry_t + jnp.squeeze(lax.slice(incl_t, (L - 1,), (L,)))
        carry_u = carry_u + jnp.squeeze(lax.slice(incl_u, (L - 1,), (L,)))
        carries_t.append(carry_t)

    bc = [jnp.broadcast_to(carries_t[c], (L,)) for c in range(7)]
    b16, b32, b48, b64, b80, b96, b112 = bc

    t_cp.wait()

    def _process(off):
        r = jax.lax.shift_right_logical(off, 8)
        col = jax.lax.bitwise_and(off, 255)
        tv = t_v[r, pl.ds(col, L)]
        m64 = b64 <= tv
        v32 = jnp.where(m64, b96, b32)
        m32 = v32 <= tv
        v16a = jnp.where(m32, b48, b16)
        v16b = jnp.where(m32, b112, b80)
        m16 = jnp.where(m64, v16b, v16a) <= tv
        idx = (jnp.where(m64, 64, 0) + jnp.where(m32, 32, 0)
               + jnp.where(m16, 16, 0))
        for s in (8, 4, 2, 1):
            cand = idx + s
            skew = cand + jax.lax.shift_right_logical(cand, 4)
            ev = plsc.load_gather(et_v, [skew])
            idx = jnp.where(ev <= tv, cand, idx)
        skew = idx + jax.lax.shift_right_logical(idx, 4)
        av = plsc.load_gather(a_v, [skew])
        bv = plsc.load_gather(b_v, [skew])
        u_v[r, pl.ds(col, L)] = av + bv * tv

    plsc.parallel_loop(0, 2048, L, unroll=2)(_process)
    pltpu.sync_copy(u_v, out_hbm.at[pl.ds(r0, 8), pl.ds(c0, 256)])


def kernel(t, l_t, l_u):
    b, s = t.shape
    mesh = plsc.VectorSubcoreMesh(core_axis_name="c", subcore_axis_name="s")
    run = pl.kernel(
        _loss_cdf_body,
        mesh=mesh,
        compiler_params=pltpu.CompilerParams(
            needs_layout_passes=False, use_tc_tiling_on_sc=True),
        out_type=jax.ShapeDtypeStruct((b, s), jnp.float32),
        scratch_types=[
            pltpu.VMEM((N_BINS,), jnp.float32),
            pltpu.VMEM((N_BINS,), jnp.float32),
            pltpu.VMEM((136,), jnp.float32),
            pltpu.VMEM((136,), jnp.float32),
            pltpu.VMEM((136,), jnp.float32),
            pltpu.VMEM((8, 256), jnp.float32),
            pltpu.VMEM((8, 256), jnp.float32),
            pltpu.SemaphoreType.DMA,
            pltpu.SemaphoreType.DMA,
        ],
    )
    return run(t, l_t, l_u)
